# baseline (speedup 1.0000x reference)
import functools

import jax
import jax.numpy as jnp
from jax import lax
from jax.experimental import pallas as pl
from jax.experimental.pallas import tpu as pltpu
from jax.experimental.pallas import tpu_sc as plsc

_C = 32


@functools.partial(jax.jit, static_argnums=(2, 3, 4))
def _gather_call(input, weight, B, V, D):
    info = plsc.get_sparse_core_info()
    NC = info.num_cores
    L = info.num_lanes
    NW = NC * info.num_subcores
    t_w = B // NW
    n_chunk = t_w // _C
    mesh = plsc.VectorSubcoreMesh(core_axis_name="c", subcore_axis_name="s")

    @functools.partial(
        pl.kernel,
        mesh=mesh,
        compiler_params=pltpu.CompilerParams(needs_layout_passes=False),
        out_type=jax.ShapeDtypeStruct((B, D), jnp.float32),
        scratch_types=[
            pltpu.VMEM((t_w,), jnp.int32),
            pltpu.VMEM((_C, D), jnp.float32),
            pltpu.VMEM((_C, D), jnp.float32),
            pltpu.SemaphoreType.DMA,
            pltpu.SemaphoreType.DMA,
        ],
    )
    def k(idx_hbm, table_hbm, out_hbm, idx_v, rbuf_a, rbuf_b, sem_a, sem_b):
        wid = lax.axis_index("s") * NC + lax.axis_index("c")
        base = wid * t_w
        pltpu.sync_copy(idx_hbm.at[pl.ds(base, t_w)], idx_v)

        def issue(cc, rbuf, sem):
            off = cc * _C
            for g in range(_C // L):
                iv = idx_v[pl.ds(off + g * L, L)]
                for j in range(L):
                    pltpu.async_copy(
                        table_hbm.at[pl.ds(iv[j], 1)],
                        rbuf.at[pl.ds(g * L + j, 1)],
                        sem,
                    )

        def consume(cc, rbuf, sem):
            for t in range(_C):
                pltpu.make_async_copy(
                    table_hbm.at[pl.ds(0, 1)], rbuf.at[pl.ds(t, 1)], sem
                ).wait()
            pltpu.sync_copy(rbuf, out_hbm.at[pl.ds(base + cc * _C, _C)])

        issue(0, rbuf_a, sem_a)

        def chunk(c, carry):
            nxt = c + 1
            pl.when(jnp.logical_and(nxt < n_chunk, (nxt & 1) == 0))(
                lambda: issue(nxt, rbuf_a, sem_a)
            )
            pl.when(jnp.logical_and(nxt < n_chunk, (nxt & 1) == 1))(
                lambda: issue(nxt, rbuf_b, sem_b)
            )
            pl.when((c & 1) == 0)(lambda: consume(c, rbuf_a, sem_a))
            pl.when((c & 1) == 1)(lambda: consume(c, rbuf_b, sem_b))
            return carry

        lax.fori_loop(0, n_chunk, chunk, 0)

    return k(input, weight)


def kernel(input, weight):
    (B,) = input.shape
    V, D = weight.shape
    return _gather_call(input, weight, B, V, D)


# speedup vs baseline: 1.3596x; 1.3596x over previous
import functools

import jax
import jax.numpy as jnp
from jax import lax
from jax.experimental import pallas as pl
from jax.experimental.pallas import tpu as pltpu
from jax.experimental.pallas import tpu_sc as plsc

_C = 32


@functools.partial(jax.jit, static_argnums=(2, 3, 4))
def _gather_call(input, weight, B, V, D):
    info = plsc.get_sparse_core_info()
    NC = info.num_cores
    L = info.num_lanes
    NW = NC * info.num_subcores
    t_w = B // NW
    n_chunk = t_w // _C
    table3 = weight.reshape(V // 8, 8, D)
    mesh = plsc.VectorSubcoreMesh(core_axis_name="c", subcore_axis_name="s")

    @functools.partial(
        pl.kernel,
        mesh=mesh,
        compiler_params=pltpu.CompilerParams(needs_layout_passes=False),
        out_type=jax.ShapeDtypeStruct((B // 8, 8, D), jnp.float32),
        scratch_types=[
            pltpu.VMEM((t_w,), jnp.int32),
            pltpu.VMEM((t_w,), jnp.int32),
            pltpu.VMEM((t_w,), jnp.int32),
            pltpu.VMEM((_C, 8, D), jnp.float32),
            pltpu.VMEM((_C, 8, D), jnp.float32),
            pltpu.VMEM((_C // 8, 8, D), jnp.float32),
            pltpu.SemaphoreType.DMA,
            pltpu.SemaphoreType.DMA,
        ],
    )
    def k(idx_hbm, table_hbm, out_hbm, idx_v, tidx_v, sub_v, gbuf_a, gbuf_b,
          obuf, sem_a, sem_b):
        wid = lax.axis_index("s") * NC + lax.axis_index("c")
        base = wid * t_w
        obase = wid * (t_w // 8)
        pltpu.sync_copy(idx_hbm.at[pl.ds(base, t_w)], idx_v)

        def split(j, carry):
            v = idx_v[pl.ds(j * L, L)]
            tidx_v[pl.ds(j * L, L)] = lax.shift_right_logical(v, 3)
            sub_v[pl.ds(j * L, L)] = lax.bitwise_and(v, 7)
            return carry

        lax.fori_loop(0, t_w // L, split, 0)

        def issue(cc, gbuf, sem):
            off = cc * _C
            for g in range(_C // L):
                tv = tidx_v[pl.ds(off + g * L, L)]
                for j in range(L):
                    pltpu.async_copy(
                        table_hbm.at[pl.ds(tv[j], 1)],
                        gbuf.at[pl.ds(g * L + j, 1)],
                        sem,
                    )

        def consume(cc, gbuf, sem):
            off = cc * _C
            for t in range(_C):
                pltpu.make_async_copy(
                    table_hbm.at[pl.ds(0, 1)], gbuf.at[pl.ds(t, 1)], sem
                ).wait()
            for g in range(_C // L):
                sv = sub_v[pl.ds(off + g * L, L)]
                for j in range(L):
                    t = g * L + j
                    tvv = jnp.full((L,), t, jnp.int32)
                    svv = jnp.full((L,), sv[j], jnp.int32)
                    for cb in range(D // L):
                        lvec = lax.iota(jnp.int32, L) + cb * L
                        vals = plsc.load_gather(gbuf, [tvv, svv, lvec])
                        obuf[t // 8, t % 8, pl.ds(cb * L, L)] = vals
            pltpu.sync_copy(
                obuf, out_hbm.at[pl.ds(obase + cc * (_C // 8), _C // 8)]
            )

        issue(0, gbuf_a, sem_a)

        def chunk(c, carry):
            nxt = c + 1
            pl.when(jnp.logical_and(nxt < n_chunk, (nxt & 1) == 0))(
                lambda: issue(nxt, gbuf_a, sem_a)
            )
            pl.when(jnp.logical_and(nxt < n_chunk, (nxt & 1) == 1))(
                lambda: issue(nxt, gbuf_b, sem_b)
            )
            pl.when((c & 1) == 0)(lambda: consume(c, gbuf_a, sem_a))
            pl.when((c & 1) == 1)(lambda: consume(c, gbuf_b, sem_b))
            return carry

        lax.fori_loop(0, n_chunk, chunk, 0)

    out3 = k(input, table3)
    return out3.reshape(B, D)


def kernel(input, weight):
    (B,) = input.shape
    V, D = weight.shape
    return _gather_call(input, weight, B, V, D)
